# baseline (speedup 1.0000x reference)
import jax
import jax.numpy as jnp
from jax.experimental import pallas as pl

_EPS = 1e-5
_K = 16


def _head_kernel(x_ref, q_ref, t_ref, o_ref):
    c = x_ref.shape[1]
    y = t_ref[...]
    acc = None
    for ci in range(c):
        part = jnp.dot(x_ref[0, ci], q_ref[ci],
                       preferred_element_type=jnp.float32)
        acc = part if acc is None else acc + part
    o_ref[0] = jnp.maximum(acc + y, 0.0)


def kernel(xyz, params):
    W, bb, g, be = params["head"][0]
    s = g / jnp.sqrt(1.0 + _EPS)
    wt = W * s[:, None]
    t = bb * s + be
    B, C, N = xyz.shape
    O = W.shape[0]
    K = _K
    G = N // K
    P = K * O
    p = jnp.arange(P)
    k_of_p = p // O
    o_of_p = p % O
    onehot = (k_of_p[None, :] == jnp.arange(K)[:, None]).astype(xyz.dtype)
    qs = onehot[None, :, :] * wt.T[:, o_of_p][:, None, :]
    tbl = t[o_of_p][None, :]
    xr = xyz.reshape(B, C, G, K)
    out = pl.pallas_call(
        _head_kernel,
        grid=(B,),
        in_specs=[
            pl.BlockSpec((1, C, G, K), lambda b: (b, 0, 0, 0)),
            pl.BlockSpec((C, K, P), lambda b: (0, 0, 0)),
            pl.BlockSpec((1, P), lambda b: (0, 0)),
        ],
        out_specs=pl.BlockSpec((1, G, P), lambda b: (b, 0, 0)),
        out_shape=jax.ShapeDtypeStruct((B, G, P), xyz.dtype),
    )(xr, qs, tbl)
    return out.reshape(B, N, O)
